# baseline (speedup 1.0000x reference)
import functools

import jax
import jax.numpy as jnp
from jax import lax
from jax.experimental import pallas as pl
from jax.experimental.pallas import tpu as pltpu
from jax.experimental.pallas import tpu_sc as plsc

_EMBED = 32
_NUM_WORKERS = 32
_ROW_BLOCK = 2048


def _sc_gather_body(s_hbm, d_hbm, table_hbm, s_out, d_out,
                    s_idx_v, d_idx_v, s_rows_v, d_rows_v, sem_s, sem_d):
  bpw = s_idx_v.shape[0]
  wid = lax.axis_index("s") * 2 + lax.axis_index("c")
  base = wid * bpw
  pltpu.sync_copy(s_hbm.at[pl.ds(base, bpw)], s_idx_v)
  pltpu.sync_copy(d_hbm.at[pl.ds(base, bpw)], d_idx_v)
  cp_s = pltpu.async_copy(table_hbm.at[s_idx_v], s_rows_v, sem_s)
  cp_d = pltpu.async_copy(table_hbm.at[d_idx_v], d_rows_v, sem_d)
  cp_s.wait()
  cp_d.wait()
  pltpu.sync_copy(s_rows_v, s_out.at[pl.ds(base, bpw)])
  pltpu.sync_copy(d_rows_v, d_out.at[pl.ds(base, bpw)])


def _sc_gather(s_id, d_id, table):
  batch = s_id.shape[0]
  bpw = batch // _NUM_WORKERS
  mesh = plsc.VectorSubcoreMesh(core_axis_name="c", subcore_axis_name="s")
  out_ty = jax.ShapeDtypeStruct((batch, _EMBED), jnp.float32)
  fn = pl.kernel(
      _sc_gather_body,
      out_type=(out_ty, out_ty),
      mesh=mesh,
      compiler_params=pltpu.CompilerParams(use_tc_tiling_on_sc=False),
      scratch_types=[
          pltpu.VMEM((bpw,), jnp.int32),
          pltpu.VMEM((bpw,), jnp.int32),
          pltpu.VMEM((bpw, _EMBED), jnp.float32),
          pltpu.VMEM((bpw, _EMBED), jnp.float32),
          pltpu.SemaphoreType.DMA,
          pltpu.SemaphoreType.DMA,
      ],
  )
  return fn(s_id, d_id, table)


def _tc_compute_body(hs_ref, hd_ref, q_ref, out_ref):
  t = jnp.dot(hs_ref[...], q_ref[...], preferred_element_type=jnp.float32)
  out_ref[...] = jnp.exp(jnp.sum(t * hd_ref[...], axis=1))


def _tc_compute(h_s, h_d, Q):
  batch = h_s.shape[0]
  grid = batch // _ROW_BLOCK
  return pl.pallas_call(
      _tc_compute_body,
      grid=(grid,),
      in_specs=[
          pl.BlockSpec((_ROW_BLOCK, _EMBED), lambda i: (i, 0)),
          pl.BlockSpec((_ROW_BLOCK, _EMBED), lambda i: (i, 0)),
          pl.BlockSpec((_EMBED, _EMBED), lambda i: (0, 0)),
      ],
      out_specs=pl.BlockSpec((_ROW_BLOCK,), lambda i: (i,)),
      out_shape=jax.ShapeDtypeStruct((batch,), jnp.float32),
  )(h_s, h_d, Q)


@jax.jit
def kernel(s_id, d_id, h_static, Q):
  h_s, h_d = _sc_gather(s_id.astype(jnp.int32), d_id.astype(jnp.int32),
                        h_static)
  return _tc_compute(h_s, h_d, Q)
